# Initial kernel scaffold; baseline (speedup 1.0000x reference)
import functools

import jax
import jax.numpy as jnp
from jax import lax
from jax.experimental import pallas as pl
from jax.experimental.pallas import tpu as pltpu
from jax.experimental.pallas import tpu_sc as plsc

EMBED_DIM = 64
IDX_PER_GATHER = 128


@functools.cache
def _build(B: int, D: int):
    info = plsc.get_sparse_core_info()
    NC, NS = info.num_cores, info.num_subcores
    NW = NC * NS
    assert B % (NW * IDX_PER_GATHER) == 0
    rows_per_w = B // NW
    IB = 8
    C = IB * IDX_PER_GATHER
    n_blocks = rows_per_w // C
    irows_per_w = rows_per_w // IDX_PER_GATHER

    mesh = plsc.VectorSubcoreMesh(core_axis_name="c", subcore_axis_name="s")

    @functools.partial(
        pl.kernel,
        mesh=mesh,
        out_type=jax.ShapeDtypeStruct((B, D), jnp.float32),
        scratch_types=[
            pltpu.VMEM((IB, IDX_PER_GATHER), jnp.int32),
            pltpu.VMEM((C, D), jnp.float32),
            pltpu.SemaphoreType.DMA,
        ],
    )
    def k(idx_hbm, table_hbm, out_hbm, idx_v, rows_v, gsem):
        wid = lax.axis_index("s") * NC + lax.axis_index("c")
        irow0 = wid * irows_per_w

        @pl.loop(0, n_blocks)
        def _(g):
            rbase = irow0 + g * IB
            pltpu.sync_copy(idx_hbm.at[pl.ds(rbase, IB)], idx_v)
            for j in range(IB):
                pltpu.async_copy(
                    table_hbm.at[idx_v.at[j]],
                    rows_v.at[pl.ds(j * IDX_PER_GATHER, IDX_PER_GATHER)],
                    gsem,
                )
            for j in range(IB):
                pltpu.make_async_copy(
                    table_hbm.at[idx_v.at[j]],
                    rows_v.at[pl.ds(j * IDX_PER_GATHER, IDX_PER_GATHER)],
                    gsem,
                ).wait()
            pltpu.sync_copy(rows_v, out_hbm.at[pl.ds(rbase * IDX_PER_GATHER, C)])

    return k


def kernel(token_ids, w):
    B = token_ids.size
    D = w.shape[-1]
    idx2d = token_ids.reshape(B // IDX_PER_GATHER, IDX_PER_GATHER).astype(jnp.int32)
    out = _build(B, D)(idx2d, w)
    return out.reshape(token_ids.shape + (D,))


# speedup vs baseline: 1.8561x; 1.8561x over previous
import functools

import jax
import jax.numpy as jnp
from jax import lax
from jax.experimental import pallas as pl
from jax.experimental.pallas import tpu as pltpu
from jax.experimental.pallas import tpu_sc as plsc

EMBED_DIM = 64
IDX_PER_GATHER = 128


@functools.cache
def _build(B: int, D: int):
    info = plsc.get_sparse_core_info()
    NC, NS = info.num_cores, info.num_subcores
    NW = NC * NS
    assert B % (NW * IDX_PER_GATHER) == 0
    rows_per_w = B // NW
    IB = 8
    C = IB * IDX_PER_GATHER
    n_blocks = rows_per_w // C
    irows_per_w = rows_per_w // IDX_PER_GATHER

    mesh = plsc.VectorSubcoreMesh(core_axis_name="c", subcore_axis_name="s")

    @functools.partial(
        pl.kernel,
        mesh=mesh,
        compiler_params=pltpu.CompilerParams(use_tc_tiling_on_sc=False),
        out_type=jax.ShapeDtypeStruct((B, D), jnp.float32),
        scratch_types=[
            pltpu.VMEM((IB, IDX_PER_GATHER), jnp.int32),
            pltpu.VMEM((C, D), jnp.float32),
            pltpu.SemaphoreType.DMA,
        ],
    )
    def k(idx_hbm, table_hbm, out_hbm, idx_v, rows_v, gsem):
        wid = lax.axis_index("s") * NC + lax.axis_index("c")
        irow0 = wid * irows_per_w

        @pl.loop(0, n_blocks)
        def _(g):
            rbase = irow0 + g * IB
            pltpu.sync_copy(idx_hbm.at[pl.ds(rbase, IB)], idx_v)
            for j in range(IB):
                pltpu.async_copy(
                    table_hbm.at[idx_v.at[j]],
                    rows_v.at[pl.ds(j * IDX_PER_GATHER, IDX_PER_GATHER)],
                    gsem,
                )
            for j in range(IB):
                pltpu.make_async_copy(
                    table_hbm.at[idx_v.at[j]],
                    rows_v.at[pl.ds(j * IDX_PER_GATHER, IDX_PER_GATHER)],
                    gsem,
                ).wait()
            pltpu.sync_copy(rows_v, out_hbm.at[pl.ds(rbase * IDX_PER_GATHER, C)])

    return k


def kernel(token_ids, w):
    B = token_ids.size
    D = w.shape[-1]
    idx2d = token_ids.reshape(B // IDX_PER_GATHER, IDX_PER_GATHER).astype(jnp.int32)
    out = _build(B, D)(idx2d, w)
    return out.reshape(token_ids.shape + (D,))


# speedup vs baseline: 1.8743x; 1.0098x over previous
import functools

import jax
import jax.numpy as jnp
from jax import lax
from jax.experimental import pallas as pl
from jax.experimental.pallas import tpu as pltpu
from jax.experimental.pallas import tpu_sc as plsc

EMBED_DIM = 64
IDX_PER_GATHER = 128
IB = 5
NBUF = 2


@functools.cache
def _build(B: int, D: int):
    info = plsc.get_sparse_core_info()
    NC, NS = info.num_cores, info.num_subcores
    NW = NC * NS
    rows_per_w = B // NW
    C = IB * IDX_PER_GATHER
    n_blocks = rows_per_w // C
    irows_per_w = rows_per_w // IDX_PER_GATHER
    assert rows_per_w % C == 0 and n_blocks % NBUF == 0
    assert B % (NW * IDX_PER_GATHER) == 0

    mesh = plsc.VectorSubcoreMesh(core_axis_name="c", subcore_axis_name="s")

    @functools.partial(
        pl.kernel,
        mesh=mesh,
        compiler_params=pltpu.CompilerParams(use_tc_tiling_on_sc=False),
        out_type=jax.ShapeDtypeStruct((B, D), jnp.float32),
        scratch_types=[
            pltpu.VMEM((irows_per_w, IDX_PER_GATHER), jnp.int32),
            pltpu.VMEM((NBUF, C, D), jnp.float32),
            pltpu.SemaphoreType.DMA((NBUF,)),
            pltpu.SemaphoreType.DMA((NBUF,)),
            pltpu.SemaphoreType.DMA,
        ],
    )
    def k(idx_hbm, table_hbm, out_hbm, idx_v, rows_v, gsem, osem, isem):
        wid = lax.axis_index("s") * NC + lax.axis_index("c")
        irow0 = wid * irows_per_w

        pltpu.async_copy(
            idx_hbm.at[pl.ds(irow0, irows_per_w)], idx_v, isem
        ).wait()

        def fire_g(g, s):
            for j in range(IB):
                pltpu.async_copy(
                    table_hbm.at[idx_v.at[g * IB + j]],
                    rows_v.at[s, pl.ds(j * IDX_PER_GATHER, IDX_PER_GATHER)],
                    gsem.at[s],
                )

        def drain_g(g, s):
            for j in range(IB):
                pltpu.make_async_copy(
                    table_hbm.at[idx_v.at[g * IB + j]],
                    rows_v.at[s, pl.ds(j * IDX_PER_GATHER, IDX_PER_GATHER)],
                    gsem.at[s],
                ).wait()

        def out_desc(g, s):
            return pltpu.make_async_copy(
                rows_v.at[s],
                out_hbm.at[pl.ds((irow0 + g * IB) * IDX_PER_GATHER, C)],
                osem.at[s],
            )

        fire_g(0, 0)
        drain_g(0, 0)
        out_desc(0, 0).start()
        fire_g(1, 1)

        @pl.loop(1, n_blocks - 1, step=NBUF)
        def _(g0):
            for b in range(NBUF):
                g = g0 + b
                s = (1 + b) % NBUF
                o = (s + 1) % NBUF
                drain_g(g, s)
                out_desc(g, s).start()
                out_desc_prev = pltpu.make_async_copy(
                    rows_v.at[o],
                    out_hbm.at[
                        pl.ds((irow0 + (g - 1) * IB) * IDX_PER_GATHER, C)
                    ],
                    osem.at[o],
                )
                out_desc_prev.wait()
                fire_g(g + 1, o)

        drain_g(n_blocks - 1, 1)
        out_desc(n_blocks - 1, 1).start()
        out_desc(n_blocks - 2, 0).wait()
        out_desc(n_blocks - 1, 1).wait()

    return k


def kernel(token_ids, w):
    B = token_ids.size
    D = w.shape[-1]
    idx2d = token_ids.reshape(B // IDX_PER_GATHER, IDX_PER_GATHER).astype(jnp.int32)
    out = _build(B, D)(idx2d, w)
    return out.reshape(token_ids.shape + (D,))
